# baseline (speedup 1.0000x reference)
import functools

import jax
import jax.numpy as jnp
from jax import lax
from jax.experimental import pallas as pl
from jax.experimental.pallas import tpu as pltpu
from jax.experimental.pallas import tpu_sc as plsc

NE = 8192
D = 256
NT = 8192
TT = 256
G = NT // TT
COMMIT_W = 0.25

_NW = 32
_BPW = NT // _NW
_CH = 128
_L = 16


def _vq_body(x_ref, w_ref, sx_ref, se_ref, enc_ref, idx_ref, loss_ref, idx_s):
    t = pl.program_id(0)

    @pl.when(t > 0)
    def _():
        iota_row = jax.lax.broadcasted_iota(jnp.int32, (1, NE), 1)
        enc_ref[...] = (iota_row == idx_s[...]).astype(jnp.float32)

    @pl.when(t < G)
    def _():
        xb = x_ref[0].reshape(D, TT)
        x2 = jnp.transpose(xb, (1, 0)) * 2.0
        mm2 = jnp.dot(x2, w_ref[...].T,
                      preferred_element_type=jnp.float32)
        d = (sx_ref[...] + se_ref[...]) - mm2
        idx = jnp.argmin(d, axis=1, keepdims=True).astype(jnp.int32)
        idx_ref[...] = jnp.transpose(idx, (1, 0)).reshape(1, 1, TT)
        idx_s[...] = idx
        dmin = jnp.min(d, axis=1, keepdims=True)

        @pl.when(t == 0)
        def _():
            loss_ref[...] = jnp.zeros((1, 1), jnp.float32)
        loss_ref[...] += jnp.sum(dmin).reshape(1, 1)


@functools.partial(
    pl.kernel,
    mesh=plsc.VectorSubcoreMesh(core_axis_name="c", subcore_axis_name="s"),
    out_type=jax.ShapeDtypeStruct((NT, D), jnp.float32),
    scratch_types=[
        pltpu.VMEM((_BPW,), jnp.int32),
        pltpu.VMEM((_CH, D), jnp.float32),
        pltpu.SemaphoreType.DMA,
    ],
)
def _sc_gather(table_hbm, idx_hbm, out_hbm, idx_v, rows_v, sem):
    wid = lax.axis_index("s") * 2 + lax.axis_index("c")
    base = wid * _BPW
    pltpu.sync_copy(idx_hbm.at[wid, 0], idx_v)
    for b in range(_BPW // _CH):
        off = base + b * _CH
        pltpu.async_copy(table_hbm.at[idx_v.at[pl.ds(b * _CH, _CH)]],
                         rows_v, sem).wait()
        pltpu.sync_copy(rows_v, out_hbm.at[pl.ds(off, _CH)])


def kernel(x, embedding_weight):
    xp = jnp.transpose(x, (0, 2, 3, 1))
    flat_x = xp.reshape(NT, D)
    sx = jnp.sum(flat_x ** 2, axis=1, keepdims=True)
    se = jnp.sum(embedding_weight ** 2, axis=1)[None, :]

    enc, idx, loss_acc = pl.pallas_call(
        _vq_body,
        grid=(G + 1,),
        in_specs=[
            pl.BlockSpec(
                (1, D, 8, 32),
                lambda t: (jnp.minimum(t, G - 1) // 4, 0,
                           jnp.minimum(t, G - 1) % 4, 0)),
            pl.BlockSpec((NE, D), lambda t: (0, 0)),
            pl.BlockSpec((TT, 1), lambda t: (jnp.minimum(t, G - 1), 0)),
            pl.BlockSpec((1, NE), lambda t: (0, 0)),
        ],
        out_specs=[
            pl.BlockSpec((TT, NE), lambda t: (jnp.maximum(t - 1, 0), 0)),
            pl.BlockSpec((1, 1, TT), lambda t: (t, 0, 0)),
            pl.BlockSpec((1, 1), lambda t: (0, 0)),
        ],
        out_shape=[
            jax.ShapeDtypeStruct((NT, NE), jnp.float32),
            jax.ShapeDtypeStruct((G + 1, 1, TT), jnp.int32),
            jax.ShapeDtypeStruct((1, 1), jnp.float32),
        ],
        scratch_shapes=[pltpu.VMEM((TT, 1), jnp.int32)],
    )(x, embedding_weight, sx, se)

    qf = _sc_gather(embedding_weight, idx)

    loss = COMMIT_W * (loss_acc[0, 0] / (NT * D))
    quantized = jnp.transpose(qf.reshape(8, 32, 32, D), (0, 3, 1, 2))
    return (loss, quantized, enc)


# speedup vs baseline: 1.1831x; 1.1831x over previous
import functools

import jax
import jax.numpy as jnp
from jax import lax
from jax.experimental import pallas as pl
from jax.experimental.pallas import tpu as pltpu
from jax.experimental.pallas import tpu_sc as plsc

NE = 8192
D = 256
NT = 8192
TT = 256
G = NT // TT
COMMIT_W = 0.25

_NW = 32
_BPW = NT // _NW
_CH = 128
_L = 16


def _vq_body(x_ref, w_ref, sx_ref, se_ref, enc_ref, idx_ref, loss_ref, idx_s):
    t = pl.program_id(0)

    @pl.when(t > 0)
    def _():
        iota_row = jax.lax.broadcasted_iota(jnp.int32, (1, NE), 1)
        enc_ref[...] = (iota_row == idx_s[...]).astype(jnp.float32)

    @pl.when(t < G)
    def _():
        mm2 = jnp.dot(x_ref[...], w_ref[...].T,
                      preferred_element_type=jnp.float32)
        d = (sx_ref[...] + se_ref[...]) - mm2
        idx = jnp.argmin(d, axis=1, keepdims=True).astype(jnp.int32)
        idx_ref[...] = jnp.transpose(idx, (1, 0)).reshape(1, 1, TT)
        idx_s[...] = idx
        dmin = jnp.min(d, axis=1, keepdims=True)

        @pl.when(t == 0)
        def _():
            loss_ref[...] = jnp.zeros((1, 1), jnp.float32)
        loss_ref[...] += jnp.sum(dmin).reshape(1, 1)


@functools.partial(
    pl.kernel,
    mesh=plsc.VectorSubcoreMesh(core_axis_name="c", subcore_axis_name="s"),
    out_type=jax.ShapeDtypeStruct((NT, D), jnp.float32),
    scratch_types=[
        pltpu.VMEM((_BPW,), jnp.int32),
        pltpu.VMEM((_CH, D), jnp.float32),
        pltpu.SemaphoreType.DMA,
    ],
)
def _sc_gather(table_hbm, idx_hbm, out_hbm, idx_v, rows_v, sem):
    wid = lax.axis_index("s") * 2 + lax.axis_index("c")
    base = wid * _BPW
    pltpu.sync_copy(idx_hbm.at[wid, 0], idx_v)
    for b in range(_BPW // _CH):
        off = base + b * _CH
        pltpu.async_copy(table_hbm.at[idx_v.at[pl.ds(b * _CH, _CH)]],
                         rows_v, sem).wait()
        pltpu.sync_copy(rows_v, out_hbm.at[pl.ds(off, _CH)])


def kernel(x, embedding_weight):
    xp = jnp.transpose(x, (0, 2, 3, 1))
    flat_x = xp.reshape(NT, D)
    sx = jnp.sum(flat_x ** 2, axis=1, keepdims=True)
    se = jnp.sum(embedding_weight ** 2, axis=1)[None, :]

    enc, idx, loss_acc = pl.pallas_call(
        _vq_body,
        grid=(G + 1,),
        in_specs=[
            pl.BlockSpec((TT, D), lambda t: (jnp.minimum(t, G - 1), 0)),
            pl.BlockSpec((NE, D), lambda t: (0, 0)),
            pl.BlockSpec((TT, 1), lambda t: (jnp.minimum(t, G - 1), 0)),
            pl.BlockSpec((1, NE), lambda t: (0, 0)),
        ],
        out_specs=[
            pl.BlockSpec((TT, NE), lambda t: (jnp.maximum(t - 1, 0), 0)),
            pl.BlockSpec((1, 1, TT), lambda t: (t, 0, 0)),
            pl.BlockSpec((1, 1), lambda t: (0, 0)),
        ],
        out_shape=[
            jax.ShapeDtypeStruct((NT, NE), jnp.float32),
            jax.ShapeDtypeStruct((G + 1, 1, TT), jnp.int32),
            jax.ShapeDtypeStruct((1, 1), jnp.float32),
        ],
        scratch_shapes=[pltpu.VMEM((TT, 1), jnp.int32)],
    )(flat_x * 2.0, embedding_weight, sx, se)

    qf = _sc_gather(embedding_weight, idx)

    loss = COMMIT_W * (loss_acc[0, 0] / (NT * D))
    quantized = jnp.transpose(qf.reshape(8, 32, 32, D), (0, 3, 1, 2))
    return (loss, quantized, enc)


# speedup vs baseline: 1.2528x; 1.0589x over previous
import functools

import jax
import jax.numpy as jnp
from jax import lax
from jax.experimental import pallas as pl
from jax.experimental.pallas import tpu as pltpu
from jax.experimental.pallas import tpu_sc as plsc

NE = 8192
D = 256
NT = 8192
TT = 256
G = NT // TT
COMMIT_W = 0.25

_NW = 32
_BPW = NT // _NW
_CH = 128
_L = 16


def _vq_body(x_ref, w_ref, sx_ref, se_ref, enc_ref, idx_ref, idx_s):
    t = pl.program_id(0)

    @pl.when(t > 0)
    def _():
        iota_row = jax.lax.broadcasted_iota(jnp.int32, (1, NE), 1)
        enc_ref[...] = (iota_row == idx_s[...]).astype(jnp.float32)

    @pl.when(t < G)
    def _():
        mm2 = jnp.dot(x_ref[...], w_ref[...].T,
                      preferred_element_type=jnp.float32)
        sxc = jnp.transpose(sx_ref[...], (1, 0))
        d = (sxc + se_ref[...]) - mm2
        idx = jnp.argmin(d, axis=1, keepdims=True).astype(jnp.int32)
        idx_ref[...] = jnp.transpose(idx, (1, 0)).reshape(1, 1, TT)
        idx_s[...] = idx


@functools.partial(
    pl.kernel,
    mesh=plsc.VectorSubcoreMesh(core_axis_name="c", subcore_axis_name="s"),
    out_type=[
        jax.ShapeDtypeStruct((NT, D), jnp.float32),
        jax.ShapeDtypeStruct((_NW, _L), jnp.float32),
    ],
    scratch_types=[
        pltpu.VMEM((_BPW,), jnp.int32),
        pltpu.VMEM((_CH, D), jnp.float32),
        pltpu.VMEM((_CH, D), jnp.float32),
        pltpu.VMEM((_L,), jnp.float32),
        pltpu.SemaphoreType.DMA,
    ],
)
def _sc_gather_loss(table_hbm, idx_hbm, x_hbm, out_hbm, losspart_hbm,
                    idx_v, rows_v, x_v, acc_v, sem):
    wid = lax.axis_index("s") * 2 + lax.axis_index("c")
    base = wid * _BPW
    pltpu.sync_copy(idx_hbm.at[wid, 0], idx_v)
    acc = jnp.zeros((_L,), jnp.float32)
    for b in range(_BPW // _CH):
        off = base + b * _CH
        pltpu.async_copy(table_hbm.at[idx_v.at[pl.ds(b * _CH, _CH)]],
                         rows_v, sem).wait()
        pltpu.sync_copy(x_hbm.at[pl.ds(off, _CH)], x_v)

        def body(r, carry):
            parts = []
            for k in range(D // _L):
                dv = rows_v[r, pl.ds(k * _L, _L)] - x_v[r, pl.ds(k * _L, _L)]
                parts.append(dv * dv)
            while len(parts) > 1:
                parts = [parts[i] + parts[i + 1]
                         for i in range(0, len(parts), 2)]
            return carry + parts[0]

        acc = lax.fori_loop(0, _CH, body, acc)
        pltpu.sync_copy(rows_v, out_hbm.at[pl.ds(off, _CH)])
    acc_v[...] = acc
    pltpu.sync_copy(acc_v, losspart_hbm.at[wid])


def kernel(x, embedding_weight):
    xp = jnp.transpose(x, (0, 2, 3, 1))
    flat_x = xp.reshape(NT, D)
    sx_row = jnp.sum(flat_x ** 2, axis=1)[None, :]
    se = jnp.sum(embedding_weight ** 2, axis=1)[None, :]

    enc, idx = pl.pallas_call(
        _vq_body,
        grid=(G + 1,),
        in_specs=[
            pl.BlockSpec((TT, D), lambda t: (jnp.minimum(t, G - 1), 0)),
            pl.BlockSpec((NE, D), lambda t: (0, 0)),
            pl.BlockSpec((1, TT), lambda t: (0, jnp.minimum(t, G - 1))),
            pl.BlockSpec((1, NE), lambda t: (0, 0)),
        ],
        out_specs=[
            pl.BlockSpec((TT, NE), lambda t: (jnp.maximum(t - 1, 0), 0)),
            pl.BlockSpec((1, 1, TT), lambda t: (t, 0, 0)),
        ],
        out_shape=[
            jax.ShapeDtypeStruct((NT, NE), jnp.float32),
            jax.ShapeDtypeStruct((G + 1, 1, TT), jnp.int32),
        ],
        scratch_shapes=[pltpu.VMEM((TT, 1), jnp.int32)],
    )(flat_x * 2.0, embedding_weight, sx_row, se)

    qf, losspart = _sc_gather_loss(embedding_weight, idx, flat_x)

    loss = COMMIT_W * (jnp.sum(losspart) / (NT * D))
    quantized = jnp.transpose(qf.reshape(8, 32, 32, D), (0, 3, 1, 2))
    return (loss, quantized, enc)


# speedup vs baseline: 1.3112x; 1.0466x over previous
import functools

import jax
import jax.numpy as jnp
from jax import lax
from jax.experimental import pallas as pl
from jax.experimental.pallas import tpu as pltpu
from jax.experimental.pallas import tpu_sc as plsc

NE = 8192
D = 256
NT = 8192
TT = 256
G = NT // TT
COMMIT_W = 0.25

_NW = 32
_BPW = NT // _NW
_CH = 128
_L = 16


def _vq_body(x_ref, w_ref, sx_ref, se_ref, enc_ref, idx_ref, idx_s):
    t = pl.program_id(0)

    @pl.when(t > 0)
    def _():
        iota_row = jax.lax.broadcasted_iota(jnp.int32, (1, NE), 1)
        enc_ref[...] = (iota_row == idx_s[...]).astype(jnp.float32)

    @pl.when(t < G)
    def _():
        mm2 = jnp.dot(x_ref[...] * 2.0, w_ref[...].T,
                      preferred_element_type=jnp.float32)
        sxc = jnp.transpose(sx_ref[...], (1, 0))
        d = (sxc + se_ref[...]) - mm2
        idx = jnp.argmin(d, axis=1, keepdims=True).astype(jnp.int32)
        idx_ref[...] = jnp.transpose(idx, (1, 0)).reshape(1, 1, TT)
        idx_s[...] = idx


@functools.partial(
    pl.kernel,
    mesh=plsc.VectorSubcoreMesh(core_axis_name="c", subcore_axis_name="s"),
    out_type=[
        jax.ShapeDtypeStruct((NT, D), jnp.float32),
        jax.ShapeDtypeStruct((_NW, _L), jnp.float32),
    ],
    scratch_types=[
        pltpu.VMEM((_BPW,), jnp.int32),
        pltpu.VMEM((_CH, D), jnp.float32),
        pltpu.VMEM((_CH, D), jnp.float32),
        pltpu.VMEM((_L,), jnp.float32),
        pltpu.SemaphoreType.DMA,
    ],
)
def _sc_gather_loss(table_hbm, idx_hbm, x_hbm, out_hbm, losspart_hbm,
                    idx_v, rows_v, x_v, acc_v, sem):
    wid = lax.axis_index("s") * 2 + lax.axis_index("c")
    base = wid * _BPW
    pltpu.sync_copy(idx_hbm.at[wid, 0], idx_v)
    acc = jnp.zeros((_L,), jnp.float32)
    for b in range(_BPW // _CH):
        off = base + b * _CH
        pltpu.async_copy(table_hbm.at[idx_v.at[pl.ds(b * _CH, _CH)]],
                         rows_v, sem).wait()
        pltpu.sync_copy(x_hbm.at[pl.ds(off, _CH)], x_v)

        def body(r, carry):
            parts = []
            for k in range(D // _L):
                dv = rows_v[r, pl.ds(k * _L, _L)] - x_v[r, pl.ds(k * _L, _L)]
                parts.append(dv * dv)
            while len(parts) > 1:
                parts = [parts[i] + parts[i + 1]
                         for i in range(0, len(parts), 2)]
            return carry + parts[0]

        acc = lax.fori_loop(0, _CH, body, acc)
        pltpu.sync_copy(rows_v, out_hbm.at[pl.ds(off, _CH)])
    acc_v[...] = acc
    pltpu.sync_copy(acc_v, losspart_hbm.at[wid])


def kernel(x, embedding_weight):
    xp = jnp.transpose(x, (0, 2, 3, 1))
    flat_x = xp.reshape(NT, D)
    sx_row = jnp.sum(flat_x ** 2, axis=1)[None, :]
    se = jnp.sum(embedding_weight ** 2, axis=1)[None, :]

    enc, idx = pl.pallas_call(
        _vq_body,
        grid=(G + 1,),
        in_specs=[
            pl.BlockSpec((TT, D), lambda t: (jnp.minimum(t, G - 1), 0)),
            pl.BlockSpec((NE, D), lambda t: (0, 0)),
            pl.BlockSpec((1, TT), lambda t: (0, jnp.minimum(t, G - 1))),
            pl.BlockSpec((1, NE), lambda t: (0, 0)),
        ],
        out_specs=[
            pl.BlockSpec((TT, NE), lambda t: (jnp.maximum(t - 1, 0), 0)),
            pl.BlockSpec((1, 1, TT), lambda t: (t, 0, 0)),
        ],
        out_shape=[
            jax.ShapeDtypeStruct((NT, NE), jnp.float32),
            jax.ShapeDtypeStruct((G + 1, 1, TT), jnp.int32),
        ],
        scratch_shapes=[pltpu.VMEM((TT, 1), jnp.int32)],
    )(flat_x, embedding_weight, sx_row, se)

    qf, losspart = _sc_gather_loss(embedding_weight, idx, flat_x)

    loss = COMMIT_W * (jnp.sum(losspart) / (NT * D))
    quantized = jnp.transpose(qf.reshape(8, 32, 32, D), (0, 3, 1, 2))
    return (loss, quantized, enc)


# speedup vs baseline: 1.3224x; 1.0085x over previous
import functools

import jax
import jax.numpy as jnp
from jax import lax
from jax.experimental import pallas as pl
from jax.experimental.pallas import tpu as pltpu
from jax.experimental.pallas import tpu_sc as plsc

NE = 8192
D = 256
NT = 8192
TT = 256
G = NT // TT
COMMIT_W = 0.25

_NW = 32
_BPW = NT // _NW
_CH = 64
_L = 16


def _vq_body(x_ref, w_ref, sx_ref, se_ref, enc_ref, idx_ref, idx_s):
    t = pl.program_id(0)

    @pl.when(t > 0)
    def _():
        iota_row = jax.lax.broadcasted_iota(jnp.int32, (1, NE), 1)
        enc_ref[...] = (iota_row == idx_s[...]).astype(jnp.float32)

    @pl.when(t < G)
    def _():
        mm2 = jnp.dot(x_ref[...] * 2.0, w_ref[...].T,
                      preferred_element_type=jnp.float32)
        sxc = jnp.transpose(sx_ref[...], (1, 0))
        d = (sxc + se_ref[...]) - mm2
        idx = jnp.argmin(d, axis=1, keepdims=True).astype(jnp.int32)
        idx_ref[...] = jnp.transpose(idx, (1, 0)).reshape(1, 1, TT)
        idx_s[...] = idx


@functools.partial(
    pl.kernel,
    mesh=plsc.VectorSubcoreMesh(core_axis_name="c", subcore_axis_name="s"),
    out_type=[
        jax.ShapeDtypeStruct((NT, D), jnp.float32),
        jax.ShapeDtypeStruct((_NW, _L), jnp.float32),
    ],
    scratch_types=[
        pltpu.VMEM((_BPW,), jnp.int32),
        pltpu.VMEM((2, _CH, D), jnp.float32),
        pltpu.VMEM((2, _CH, D), jnp.float32),
        pltpu.VMEM((_L,), jnp.float32),
        pltpu.SemaphoreType.DMA,
        pltpu.SemaphoreType.DMA,
        pltpu.SemaphoreType.DMA,
        pltpu.SemaphoreType.DMA,
    ],
)
def _sc_gather_loss(table_hbm, idx_hbm, x_hbm, out_hbm, losspart_hbm,
                    idx_v, rows_v, x_v, acc_v, gs0, gs1, xs0, xs1):
    wid = lax.axis_index("s") * 2 + lax.axis_index("c")
    base = wid * _BPW
    nch = _BPW // _CH
    gsems = [gs0, gs1]
    xsems = [xs0, xs1]
    pltpu.sync_copy(idx_hbm.at[wid, 0], idx_v)

    def start(b):
        buf = b % 2
        g = pltpu.async_copy(table_hbm.at[idx_v.at[pl.ds(b * _CH, _CH)]],
                             rows_v.at[buf], gsems[buf])
        xc = pltpu.async_copy(x_hbm.at[pl.ds(base + b * _CH, _CH)],
                              x_v.at[buf], xsems[buf])
        return g, xc

    acc = jnp.zeros((_L,), jnp.float32)
    pend = start(0)
    for b in range(nch):
        buf = b % 2
        pend[0].wait()
        pend[1].wait()
        if b + 1 < nch:
            pend = start(b + 1)

        def body(r, carry):
            parts = []
            for k in range(D // _L):
                dv = (rows_v[buf, r, pl.ds(k * _L, _L)]
                      - x_v[buf, r, pl.ds(k * _L, _L)])
                parts.append(dv * dv)
            while len(parts) > 1:
                parts = [parts[i] + parts[i + 1]
                         for i in range(0, len(parts), 2)]
            return carry + parts[0]

        acc = lax.fori_loop(0, _CH, body, acc)
        pltpu.sync_copy(rows_v.at[buf], out_hbm.at[pl.ds(base + b * _CH, _CH)])
    acc_v[...] = acc
    pltpu.sync_copy(acc_v, losspart_hbm.at[wid])


def kernel(x, embedding_weight):
    xp = jnp.transpose(x, (0, 2, 3, 1))
    flat_x = xp.reshape(NT, D)
    sx_row = jnp.sum(flat_x ** 2, axis=1)[None, :]
    se = jnp.sum(embedding_weight ** 2, axis=1)[None, :]

    enc, idx = pl.pallas_call(
        _vq_body,
        grid=(G + 1,),
        in_specs=[
            pl.BlockSpec((TT, D), lambda t: (jnp.minimum(t, G - 1), 0)),
            pl.BlockSpec((NE, D), lambda t: (0, 0)),
            pl.BlockSpec((1, TT), lambda t: (0, jnp.minimum(t, G - 1))),
            pl.BlockSpec((1, NE), lambda t: (0, 0)),
        ],
        out_specs=[
            pl.BlockSpec((TT, NE), lambda t: (jnp.maximum(t - 1, 0), 0)),
            pl.BlockSpec((1, 1, TT), lambda t: (t, 0, 0)),
        ],
        out_shape=[
            jax.ShapeDtypeStruct((NT, NE), jnp.float32),
            jax.ShapeDtypeStruct((G + 1, 1, TT), jnp.int32),
        ],
        scratch_shapes=[pltpu.VMEM((TT, 1), jnp.int32)],
    )(flat_x, embedding_weight, sx_row, se)

    qf, losspart = _sc_gather_loss(embedding_weight, idx, flat_x)

    loss = COMMIT_W * (jnp.sum(losspart) / (NT * D))
    quantized = jnp.transpose(qf.reshape(8, 32, 32, D), (0, 3, 1, 2))
    return (loss, quantized, enc)
